# Initial kernel scaffold; baseline (speedup 1.0000x reference)
import jax
import jax.numpy as jnp
from jax.experimental import pallas as pl


def kernel(x, edge_index, lin_W0, lin_b0, agg_W0, agg_b0, lin_W1, lin_b1, agg_W1, agg_b1, post_W1, post_b1, post_W2, post_b2):
    raise NotImplementedError("write your pallas kernel here")


# speedup vs baseline: 3.6532x; 3.6532x over previous
import jax
import jax.numpy as jnp
from jax import lax
from jax.experimental import pallas as pl
from jax.experimental.pallas import tpu as pltpu
from jax.experimental.pallas import tpu_sc as plsc

_N = 10000
_E = 320000
_D = 128
_NCORES = 2
_NSUB = 16
_NW = _NCORES * _NSUB
_K = 128
_CHUNKS = -(-_E // (_NW * _K))
_PER_W = _CHUNKS * _K
_E_PAD = _PER_W * _NW
_N_PAD = 10112
_RPT = _N_PAD // _NSUB

_HIGH = lax.Precision.HIGHEST


def _make_sc_agg(width):
    mesh = plsc.VectorSubcoreMesh(core_axis_name="c", subcore_axis_name="s")

    def body(m_hbm, src_hbm, dst_hbm, zs_hbm, s_out,
             src_v, dst_v, rows_v, s_sh, sem):
        c = lax.axis_index("c")
        s = lax.axis_index("s")
        wid = c * _NSUB + s
        r0 = pl.multiple_of(s * _RPT, 8)
        pltpu.sync_copy(zs_hbm.at[pl.ds(r0, _RPT)], s_sh.at[pl.ds(r0, _RPT)])
        plsc.subcore_barrier()

        base0 = wid * _PER_W

        def step(i, carry):
            base = base0 + i * _K
            pltpu.sync_copy(src_hbm.at[pl.ds(base, _K)], src_v)
            pltpu.sync_copy(dst_hbm.at[pl.ds(base, _K)], dst_v)
            pltpu.async_copy(m_hbm.at[src_v], rows_v, sem).wait()
            pltpu.sync_copy(rows_v, s_sh.at[dst_v], add=True)
            return carry

        lax.fori_loop(0, _CHUNKS, step, 0)
        plsc.subcore_barrier()
        o0 = pl.multiple_of(c * _N_PAD + r0, 8)
        pltpu.sync_copy(s_sh.at[pl.ds(r0, _RPT)], s_out.at[pl.ds(o0, _RPT)])

    return pl.kernel(
        body,
        out_type=[jax.ShapeDtypeStruct((_NCORES * _N_PAD, width), jnp.float32)],
        mesh=mesh,
        scratch_types=[
            pltpu.VMEM((_K,), jnp.int32),
            pltpu.VMEM((_K,), jnp.int32),
            pltpu.VMEM((_K, width), jnp.float32),
            pltpu.VMEM_SHARED((_N_PAD, width), jnp.float32),
            pltpu.SemaphoreType.DMA,
        ],
    )


def _make_sc_cnt():
    mesh = plsc.VectorSubcoreMesh(core_axis_name="c", subcore_axis_name="s")

    def body(dst_hbm, zs_hbm, ones_hbm, c_out, dst_v, ones_v, c_sh):
        c = lax.axis_index("c")
        s = lax.axis_index("s")
        wid = c * _NSUB + s
        r0 = pl.multiple_of(s * _RPT, 8)
        pltpu.sync_copy(zs_hbm.at[pl.ds(r0, _RPT)], c_sh.at[pl.ds(r0, _RPT)])
        pltpu.sync_copy(ones_hbm, ones_v)
        plsc.subcore_barrier()

        base0 = wid * _PER_W

        def step(i, carry):
            base = base0 + i * _K
            pltpu.sync_copy(dst_hbm.at[pl.ds(base, _K)], dst_v)
            pltpu.sync_copy(ones_v, c_sh.at[dst_v], add=True)
            return carry

        lax.fori_loop(0, _CHUNKS, step, 0)
        plsc.subcore_barrier()
        o0 = pl.multiple_of(c * _N_PAD + r0, 8)
        pltpu.sync_copy(c_sh.at[pl.ds(r0, _RPT)], c_out.at[pl.ds(o0, _RPT)])

    return pl.kernel(
        body,
        out_type=[jax.ShapeDtypeStruct((_NCORES * _N_PAD, _D), jnp.float32)],
        mesh=mesh,
        scratch_types=[
            pltpu.VMEM((_K,), jnp.int32),
            pltpu.VMEM((_K, _D), jnp.float32),
            pltpu.VMEM_SHARED((_N_PAD, _D), jnp.float32),
        ],
    )


_SC_CACHE = {}


def _sc_agg(m, src, dst, zs):
    if "agg" not in _SC_CACHE:
        _SC_CACHE["agg"] = _make_sc_agg(_D)
    return _SC_CACHE["agg"](m, src, dst, zs)[0]


def _sc_cnt(dst, zs, ones):
    if "cnt" not in _SC_CACHE:
        _SC_CACHE["cnt"] = _make_sc_cnt()
    return _SC_CACHE["cnt"](dst, zs, ones)[0]


_BN = 2000


def _row_spec(w):
    return pl.BlockSpec((_BN, w), lambda i: (i, 0))


def _full_spec(shape):
    return pl.BlockSpec(shape, lambda i: tuple(0 for _ in shape))


def _dot(a, b):
    return jnp.dot(a, b, preferred_element_type=jnp.float32, precision=_HIGH)


def _msg_kernel(x_ref, w_ref, b_ref, o_ref):
    o_ref[...] = jnp.maximum(_dot(x_ref[...], w_ref[...]) + b_ref[...], 0.0)


def _tc_msg(x, w, b):
    return pl.pallas_call(
        _msg_kernel,
        grid=(_N // _BN,),
        in_specs=[_row_spec(_D), _full_spec((_D, _D)), _full_spec((1, _D))],
        out_specs=_row_spec(_D),
        out_shape=jax.ShapeDtypeStruct((_N, _D), jnp.float32),
    )(x, w, b)


def _mid_kernel(s0_ref, s1_ref, c0_ref, c1_ref, x_ref,
                awm_ref, awx_ref, ab_ref, lw_ref, lb_ref,
                h_ref, m_ref):
    cnt = c0_ref[...] + c1_ref[...]
    mean = (s0_ref[...] + s1_ref[...]) / jnp.maximum(cnt, 1.0)
    h = jnp.maximum(_dot(mean, awm_ref[...]) + _dot(x_ref[...], awx_ref[...])
                    + ab_ref[...], 0.0)
    nrm = jnp.maximum(jnp.sqrt(jnp.sum(h * h, axis=-1, keepdims=True)), 1e-12)
    h = h / nrm
    h_ref[...] = h
    m_ref[...] = jnp.maximum(_dot(h, lw_ref[...]) + lb_ref[...], 0.0)


def _tc_mid(s0, s1, c0, c1, x, awm, awx, ab, lw, lb):
    return pl.pallas_call(
        _mid_kernel,
        grid=(_N // _BN,),
        in_specs=[_row_spec(_D), _row_spec(_D), _row_spec(1), _row_spec(1),
                  _row_spec(_D), _full_spec((_D, _D)), _full_spec((_D, _D)),
                  _full_spec((1, _D)), _full_spec((_D, _D)), _full_spec((1, _D))],
        out_specs=[_row_spec(_D), _row_spec(_D)],
        out_shape=[jax.ShapeDtypeStruct((_N, _D), jnp.float32),
                   jax.ShapeDtypeStruct((_N, _D), jnp.float32)],
    )(s0, s1, c0, c1, x, awm, awx, ab, lw, lb)


def _fin_kernel(s0_ref, s1_ref, c0_ref, c1_ref, h_ref,
                awm_ref, awh_ref, ab_ref, pw1_ref, pb1_ref, pw2_ref, pb2_ref,
                o_ref):
    cnt = c0_ref[...] + c1_ref[...]
    mean = (s0_ref[...] + s1_ref[...]) / jnp.maximum(cnt, 1.0)
    h2 = jnp.maximum(_dot(mean, awm_ref[...]) + _dot(h_ref[...], awh_ref[...])
                     + ab_ref[...], 0.0)
    nrm = jnp.maximum(jnp.sqrt(jnp.sum(h2 * h2, axis=-1, keepdims=True)), 1e-12)
    h2 = h2 / nrm
    g = _dot(h2, pw1_ref[...]) + pb1_ref[...]
    o = _dot(g, pw2_ref[...]) + pb2_ref[...]
    mx = jnp.max(o, axis=-1, keepdims=True)
    z = o - mx
    o_ref[...] = z - jnp.log(jnp.sum(jnp.exp(z), axis=-1, keepdims=True))


def _tc_fin(s0, s1, c0, c1, h, awm, awh, ab, pw1, pb1, pw2, pb2):
    return pl.pallas_call(
        _fin_kernel,
        grid=(_N // _BN,),
        in_specs=[_row_spec(_D), _row_spec(_D), _row_spec(1), _row_spec(1),
                  _row_spec(_D), _full_spec((_D, _D)), _full_spec((_D, _D)),
                  _full_spec((1, _D)), _full_spec((_D, _D)), _full_spec((1, _D)),
                  _full_spec((_D, _D)), _full_spec((1, _D))],
        out_specs=_row_spec(_D),
        out_shape=jax.ShapeDtypeStruct((_N, _D), jnp.float32),
    )(s0, s1, c0, c1, h, awm, awh, ab, pw1, pb1, pw2, pb2)


def kernel(x, edge_index, lin_W0, lin_b0, agg_W0, agg_b0,
           lin_W1, lin_b1, agg_W1, agg_b1,
           post_W1, post_b1, post_W2, post_b2):
    d_out = post_W2.shape[1]

    src = edge_index[0].astype(jnp.int32)
    dst = edge_index[1].astype(jnp.int32)
    pad = _E_PAD - _E
    src_p = jnp.concatenate([src, jnp.zeros((pad,), jnp.int32)])
    dst_p = jnp.concatenate([dst, jnp.full((pad,), _N, jnp.int32)])
    zeros_s = jnp.zeros((_N_PAD, _D), jnp.float32)
    ones = jnp.ones((_K, _D), jnp.float32)

    lb0 = lin_b0.reshape(1, _D)
    ab0 = agg_b0.reshape(1, _D)
    lb1 = lin_b1.reshape(1, _D)
    ab1 = agg_b1.reshape(1, _D)
    pb1 = post_b1.reshape(1, _D)
    aw0m, aw0x = agg_W0[:_D], agg_W0[_D:]
    aw1m, aw1h = agg_W1[:_D], agg_W1[_D:]
    pw2 = jnp.zeros((_D, _D), jnp.float32).at[:, :d_out].set(post_W2)
    pb2 = jnp.full((1, _D), -1e30, jnp.float32).at[0, :d_out].set(post_b2)

    c_p = _sc_cnt(dst_p, zeros_s, ones).reshape(_NCORES, _N_PAD, _D)
    c0, c1 = c_p[0, :_N, :1], c_p[1, :_N, :1]
    m0 = _tc_msg(x, lin_W0, lb0)
    s_p = _sc_agg(m0, src_p, dst_p, zeros_s).reshape(_NCORES, _N_PAD, _D)
    s0, s1 = s_p[0, :_N], s_p[1, :_N]
    h, m1 = _tc_mid(s0, s1, c0, c1, x, aw0m, aw0x, ab0, lin_W1, lb1)

    t_p = _sc_agg(m1, src_p, dst_p, zeros_s).reshape(_NCORES, _N_PAD, _D)
    o = _tc_fin(t_p[0, :_N], t_p[1, :_N], c0, c1, h,
                aw1m, aw1h, ab1, post_W1, pb1, pw2, pb2)
    return o[:, :d_out]


# speedup vs baseline: 3.7392x; 1.0235x over previous
import jax
import jax.numpy as jnp
from jax import lax
from jax.experimental import pallas as pl
from jax.experimental.pallas import tpu as pltpu
from jax.experimental.pallas import tpu_sc as plsc

_N = 10000
_E = 320000
_D = 128
_NCORES = 2
_NSUB = 16
_NW = _NCORES * _NSUB
_K = 128
_NB = 2
_CHUNKS = 80
_PER_W = _CHUNKS * _K
_E_PAD = _PER_W * _NW
_N_PAD = 10112
_RPT = _N_PAD // _NSUB

_HIGH = lax.Precision.HIGHEST


def _make_sc_agg(width):
    mesh = plsc.VectorSubcoreMesh(core_axis_name="c", subcore_axis_name="s")

    def body(m_hbm, src_hbm, dst_hbm, zs_hbm, s_out,
             srcr_0, srcr_1, rows_0, rows_1, dst_v, s_sh, *sems):
        srcr = [srcr_0, srcr_1]
        rows = [rows_0, rows_1]
        isems = sems[:_NB]
        gsems = sems[_NB:2 * _NB]
        ssems = sems[2 * _NB:]
        c = lax.axis_index("c")
        s = lax.axis_index("s")
        wid = c * _NSUB + s
        r0 = pl.multiple_of(s * _RPT, 8)
        pltpu.sync_copy(dst_hbm.at[wid], dst_v)
        pltpu.sync_copy(zs_hbm.at[pl.ds(r0, _RPT)], s_sh.at[pl.ds(r0, _RPT)])
        plsc.subcore_barrier()

        def fire_idx(i, b):
            pltpu.async_copy(src_hbm.at[wid].at[i], srcr[b], isems[b])

        def wait_idx(b):
            pltpu.make_async_copy(
                src_hbm.at[0].at[0], srcr[b], isems[b]).wait()

        def fire_gather(b):
            pltpu.async_copy(m_hbm.at[srcr[b]], rows[b], gsems[b])

        def drain_gather(b):
            pltpu.make_async_copy(
                m_hbm.at[pl.ds(0, _K)], rows[b], gsems[b]).wait()

        def fire_scatter(i, b):
            pltpu.async_copy(rows[b], s_sh.at[dst_v.at[i]], ssems[b],
                             add=True)

        def drain_scatter(b):
            pltpu.make_async_copy(
                m_hbm.at[pl.ds(0, _K)], rows[b], ssems[b]).wait()

        for b in range(_NB):
            fire_idx(b, b)

        def outer(j, carry):
            for b in range(_NB):
                i = j * _NB + b

                @pl.when(j > 0)
                def _():
                    drain_scatter(b)

                wait_idx(b)
                fire_gather(b)
                drain_gather(b)
                fire_scatter(i, b)

                @pl.when(i + _NB < _CHUNKS)
                def _():
                    fire_idx(i + _NB, b)
            return carry

        lax.fori_loop(0, _CHUNKS // _NB, outer, 0)
        for b in range(_NB):
            drain_scatter(b)
        plsc.subcore_barrier()
        o0 = pl.multiple_of(c * _N_PAD + r0, 8)
        pltpu.sync_copy(s_sh.at[pl.ds(r0, _RPT)], s_out.at[pl.ds(o0, _RPT)])

    return pl.kernel(
        body,
        out_type=[jax.ShapeDtypeStruct((_NCORES * _N_PAD, width), jnp.float32)],
        mesh=mesh,
        scratch_types=[
            pltpu.VMEM((_K,), jnp.int32),
            pltpu.VMEM((_K,), jnp.int32),
            pltpu.VMEM((_K, width), jnp.float32),
            pltpu.VMEM((_K, width), jnp.float32),
            pltpu.VMEM((_CHUNKS, _K), jnp.int32),
            pltpu.VMEM_SHARED((_N_PAD, width), jnp.float32),
        ] + [pltpu.SemaphoreType.DMA] * (3 * _NB),
    )


def _make_sc_cnt():
    mesh = plsc.VectorSubcoreMesh(core_axis_name="c", subcore_axis_name="s")

    def body(dst_hbm, zs_hbm, ones_hbm, c_out, dst_v, ones_v, c_sh):
        c = lax.axis_index("c")
        s = lax.axis_index("s")
        wid = c * _NSUB + s
        r0 = pl.multiple_of(s * _RPT, 8)
        pltpu.sync_copy(dst_hbm.at[wid], dst_v)
        pltpu.sync_copy(zs_hbm.at[pl.ds(r0, _RPT)], c_sh.at[pl.ds(r0, _RPT)])
        pltpu.sync_copy(ones_hbm, ones_v)
        plsc.subcore_barrier()

        def step(i, carry):
            pltpu.sync_copy(ones_v, c_sh.at[dst_v.at[i]], add=True)
            return carry

        lax.fori_loop(0, _CHUNKS, step, 0)
        plsc.subcore_barrier()
        o0 = pl.multiple_of(c * _N_PAD + r0, 8)
        pltpu.sync_copy(c_sh.at[pl.ds(r0, _RPT)], c_out.at[pl.ds(o0, _RPT)])

    return pl.kernel(
        body,
        out_type=[jax.ShapeDtypeStruct((_NCORES * _N_PAD, _D), jnp.float32)],
        mesh=mesh,
        scratch_types=[
            pltpu.VMEM((_CHUNKS, _K), jnp.int32),
            pltpu.VMEM((_K, _D), jnp.float32),
            pltpu.VMEM_SHARED((_N_PAD, _D), jnp.float32),
        ],
    )


_SC_CACHE = {}


def _sc_agg(m, src, dst, zs):
    if "agg" not in _SC_CACHE:
        _SC_CACHE["agg"] = _make_sc_agg(_D)
    return _SC_CACHE["agg"](m, src, dst, zs)[0]


def _sc_cnt(dst, zs, ones):
    if "cnt" not in _SC_CACHE:
        _SC_CACHE["cnt"] = _make_sc_cnt()
    return _SC_CACHE["cnt"](dst, zs, ones)[0]


_BN = 2000


def _row_spec(w):
    return pl.BlockSpec((_BN, w), lambda i: (i, 0))


def _full_spec(shape):
    return pl.BlockSpec(shape, lambda i: tuple(0 for _ in shape))


def _dot(a, b):
    return jnp.dot(a, b, preferred_element_type=jnp.float32, precision=_HIGH)


def _msg_kernel(x_ref, w_ref, b_ref, o_ref):
    o_ref[...] = jnp.maximum(_dot(x_ref[...], w_ref[...]) + b_ref[...], 0.0)


def _tc_msg(x, w, b):
    return pl.pallas_call(
        _msg_kernel,
        grid=(_N // _BN,),
        in_specs=[_row_spec(_D), _full_spec((_D, _D)), _full_spec((1, _D))],
        out_specs=_row_spec(_D),
        out_shape=jax.ShapeDtypeStruct((_N, _D), jnp.float32),
    )(x, w, b)


def _mid_kernel(s0_ref, s1_ref, c0_ref, c1_ref, x_ref,
                awm_ref, awx_ref, ab_ref, lw_ref, lb_ref,
                h_ref, m_ref):
    cnt = c0_ref[...] + c1_ref[...]
    mean = (s0_ref[...] + s1_ref[...]) / jnp.maximum(cnt, 1.0)
    h = jnp.maximum(_dot(mean, awm_ref[...]) + _dot(x_ref[...], awx_ref[...])
                    + ab_ref[...], 0.0)
    nrm = jnp.maximum(jnp.sqrt(jnp.sum(h * h, axis=-1, keepdims=True)), 1e-12)
    h = h / nrm
    h_ref[...] = h
    m_ref[...] = jnp.maximum(_dot(h, lw_ref[...]) + lb_ref[...], 0.0)


def _tc_mid(s0, s1, c0, c1, x, awm, awx, ab, lw, lb):
    return pl.pallas_call(
        _mid_kernel,
        grid=(_N // _BN,),
        in_specs=[_row_spec(_D), _row_spec(_D), _row_spec(1), _row_spec(1),
                  _row_spec(_D), _full_spec((_D, _D)), _full_spec((_D, _D)),
                  _full_spec((1, _D)), _full_spec((_D, _D)), _full_spec((1, _D))],
        out_specs=[_row_spec(_D), _row_spec(_D)],
        out_shape=[jax.ShapeDtypeStruct((_N, _D), jnp.float32),
                   jax.ShapeDtypeStruct((_N, _D), jnp.float32)],
    )(s0, s1, c0, c1, x, awm, awx, ab, lw, lb)


def _fin_kernel(s0_ref, s1_ref, c0_ref, c1_ref, h_ref,
                awm_ref, awh_ref, ab_ref, pw1_ref, pb1_ref, pw2_ref, pb2_ref,
                o_ref):
    cnt = c0_ref[...] + c1_ref[...]
    mean = (s0_ref[...] + s1_ref[...]) / jnp.maximum(cnt, 1.0)
    h2 = jnp.maximum(_dot(mean, awm_ref[...]) + _dot(h_ref[...], awh_ref[...])
                     + ab_ref[...], 0.0)
    nrm = jnp.maximum(jnp.sqrt(jnp.sum(h2 * h2, axis=-1, keepdims=True)), 1e-12)
    h2 = h2 / nrm
    g = _dot(h2, pw1_ref[...]) + pb1_ref[...]
    o = _dot(g, pw2_ref[...]) + pb2_ref[...]
    mx = jnp.max(o, axis=-1, keepdims=True)
    z = o - mx
    o_ref[...] = z - jnp.log(jnp.sum(jnp.exp(z), axis=-1, keepdims=True))


def _tc_fin(s0, s1, c0, c1, h, awm, awh, ab, pw1, pb1, pw2, pb2):
    return pl.pallas_call(
        _fin_kernel,
        grid=(_N // _BN,),
        in_specs=[_row_spec(_D), _row_spec(_D), _row_spec(1), _row_spec(1),
                  _row_spec(_D), _full_spec((_D, _D)), _full_spec((_D, _D)),
                  _full_spec((1, _D)), _full_spec((_D, _D)), _full_spec((1, _D)),
                  _full_spec((_D, _D)), _full_spec((1, _D))],
        out_specs=_row_spec(_D),
        out_shape=jax.ShapeDtypeStruct((_N, _D), jnp.float32),
    )(s0, s1, c0, c1, h, awm, awh, ab, pw1, pb1, pw2, pb2)


def kernel(x, edge_index, lin_W0, lin_b0, agg_W0, agg_b0,
           lin_W1, lin_b1, agg_W1, agg_b1,
           post_W1, post_b1, post_W2, post_b2):
    d_out = post_W2.shape[1]

    src = edge_index[0].astype(jnp.int32)
    dst = edge_index[1].astype(jnp.int32)
    pad = _E_PAD - _E
    src_p = jnp.concatenate(
        [src, jnp.zeros((pad,), jnp.int32)]).reshape(_NW, _CHUNKS, _K)
    dst_p = jnp.concatenate(
        [dst, jnp.full((pad,), _N, jnp.int32)]).reshape(_NW, _CHUNKS, _K)
    zeros_s = jnp.zeros((_N_PAD, _D), jnp.float32)
    ones = jnp.ones((_K, _D), jnp.float32)

    lb0 = lin_b0.reshape(1, _D)
    ab0 = agg_b0.reshape(1, _D)
    lb1 = lin_b1.reshape(1, _D)
    ab1 = agg_b1.reshape(1, _D)
    pb1 = post_b1.reshape(1, _D)
    aw0m, aw0x = agg_W0[:_D], agg_W0[_D:]
    aw1m, aw1h = agg_W1[:_D], agg_W1[_D:]
    pw2 = jnp.zeros((_D, _D), jnp.float32).at[:, :d_out].set(post_W2)
    pb2 = jnp.full((1, _D), -1e30, jnp.float32).at[0, :d_out].set(post_b2)

    c_p = _sc_cnt(dst_p, zeros_s, ones).reshape(_NCORES, _N_PAD, _D)
    c0, c1 = c_p[0, :_N, :1], c_p[1, :_N, :1]
    m0 = _tc_msg(x, lin_W0, lb0)
    s_p = _sc_agg(m0, src_p, dst_p, zeros_s).reshape(_NCORES, _N_PAD, _D)
    s0, s1 = s_p[0, :_N], s_p[1, :_N]
    h, m1 = _tc_mid(s0, s1, c0, c1, x, aw0m, aw0x, ab0, lin_W1, lb1)

    t_p = _sc_agg(m1, src_p, dst_p, zeros_s).reshape(_NCORES, _N_PAD, _D)
    o = _tc_fin(t_p[0, :_N], t_p[1, :_N], c0, c1, h,
                aw1m, aw1h, ab1, post_W1, pb1, pw2, pb2)
    return o[:, :d_out]
